# Initial kernel scaffold; baseline (speedup 1.0000x reference)
import jax
import jax.numpy as jnp
from jax.experimental import pallas as pl


def kernel(input, label):
    raise NotImplementedError("write your pallas kernel here")


# speedup vs baseline: 17.7338x; 17.7338x over previous
import functools

import jax
import jax.numpy as jnp
from jax import lax
from jax.experimental import pallas as pl
from jax.experimental.pallas import tpu as pltpu
from jax.experimental.pallas import tpu_sc as plsc

NC = 2
NS = 16
L = 16
NSLICE = 8
NELEM = 1 << 20
K = 524288
PER_TILE = NELEM // NS
CHUNK = 8192
NB = 512
COARSE_SHIFT = 22
FINE_SHIFT = 13
SLICES_PER_CORE = NSLICE // NC

OFF_CNT = 0
OFF_SUM = NS * NB
OFF_ACC = 2 * NS * NB
OFF_BC = 2 * NS * NB + NS * L
SH_WORDS = OFF_BC + 16


def _suffix_find(cnt_ref, sum_ref, thresh, lanes):
  def body(i, carry):
    found, bst, sac, sas, c_run, s_run = carry
    g = 31 - i
    c = cnt_ref[pl.ds(g * 16, 16)]
    s = sum_ref[pl.ds(g * 16, 16)]
    suffc = lax.rev(plsc.cumsum(lax.rev(c, (0,))), (0,))
    suffs = lax.rev(plsc.cumsum(lax.rev(s, (0,))), (0,))
    incl = c_run + suffc
    mask = incl >= thresh
    pc = jnp.max(plsc.all_reduce_population_count(mask))
    found_here = jnp.logical_and(pc > 0, jnp.logical_not(found))
    p0 = pc - 1
    onehot = lanes == p0
    cp = jnp.sum(jnp.where(onehot, c, jnp.float32(0)))
    sufcp = jnp.sum(jnp.where(onehot, suffc, jnp.float32(0)))
    sp = jnp.sum(jnp.where(onehot, s, jnp.float32(0)))
    sufsp = jnp.sum(jnp.where(onehot, suffs, jnp.float32(0)))
    bst = jnp.where(found_here, g * 16 + p0, bst)
    sac = jnp.where(found_here, c_run + sufcp - cp, sac)
    sas = jnp.where(found_here, s_run + sufsp - sp, sas)
    c_run = c_run + jnp.sum(c)
    s_run = s_run + jnp.sum(s)
    found = jnp.logical_or(found, pc > 0)
    return found, bst, sac, sas, c_run, s_run

  init = (jnp.bool_(False), jnp.int32(0), jnp.float32(0), jnp.float32(0),
          jnp.float32(0), jnp.float32(0))
  out = lax.fori_loop(0, 32, body, init)
  return out[1], out[2], out[3]


def _loss_program(x_hbm, y_hbm, out_hbm,
                  d_v, y_v, h1_v, fcnt_v, fsum_v,
                  c1_v, cs_v, row_c, row_s, tmp_f, sh):
  c = lax.axis_index("c")
  s = lax.axis_index("s")
  lanes = jnp.arange(16, dtype=jnp.int32)
  lane_base = lanes * NB
  ones_f = jnp.ones((16,), jnp.float32)
  zeros_f = jnp.zeros((16,), jnp.float32)

  def per_slice(si, _):
    slice_idx = c * SLICES_PER_CORE + si
    base = slice_idx * NELEM + s * PER_TILE

    def zero_h1(i, _):
      h1_v[pl.ds(i * 16, 16)] = zeros_f
      return 0
    lax.fori_loop(0, (L * NB) // 16, zero_h1, 0)

    pltpu.sync_copy(x_hbm.at[pl.ds(base, PER_TILE)], d_v)

    def chunk_body(j, _):
      pltpu.sync_copy(y_hbm.at[pl.ds(base + j * CHUNK, CHUNK)], y_v)

      def vec_body(i, _):
        off = j * CHUNK + i * 16
        xv = d_v[pl.ds(off, 16)]
        yv = y_v[pl.ds(i * 16, 16)]
        dv = xv - yv
        dv = dv * dv
        d_v[pl.ds(off, 16)] = dv
        u = plsc.bitcast(dv, jnp.int32)
        b = lax.shift_right_logical(u, COARSE_SHIFT)
        plsc.addupdate_scatter(h1_v, [lane_base + b], ones_f)
        return 0
      lax.fori_loop(0, CHUNK // 16, vec_body, 0)
      return 0
    lax.fori_loop(0, PER_TILE // CHUNK, chunk_body, 0)

    def merge_lanes_1(g, _):
      acc = zeros_f
      for l in range(L):
        acc = acc + h1_v[pl.ds(l * NB + g * 16, 16)]
      c1_v[pl.ds(g * 16, 16)] = acc
      return 0
    lax.fori_loop(0, NB // 16, merge_lanes_1, 0)

    pltpu.sync_copy(c1_v, sh.at[pl.ds(OFF_CNT + s * NB, NB)])
    plsc.subcore_barrier()

    @pl.when(s == 0)
    def _():
      def zero_c1(i, _):
        c1_v[pl.ds(i * 16, 16)] = zeros_f
        cs_v[pl.ds(i * 16, 16)] = zeros_f
        return 0
      lax.fori_loop(0, NB // 16, zero_c1, 0)
      for t in range(NS):
        pltpu.sync_copy(sh.at[pl.ds(OFF_CNT + t * NB, NB)], row_c)

        def add_row(g, _):
          c1_v[pl.ds(g * 16, 16)] = c1_v[pl.ds(g * 16, 16)] + row_c[pl.ds(g * 16, 16)]
          return 0
        lax.fori_loop(0, NB // 16, add_row, 0)
      bst, sac, _ = _suffix_find(c1_v, cs_v, jnp.float32(K), lanes)
      rem = jnp.float32(K) - sac
      bc = (jnp.where(lanes == 0, bst.astype(jnp.float32), jnp.float32(0))
            + jnp.where(lanes == 1, rem, jnp.float32(0)))
      tmp_f[...] = bc
      pltpu.sync_copy(tmp_f, sh.at[pl.ds(OFF_BC, 16)])
    plsc.subcore_barrier()

    pltpu.sync_copy(sh.at[pl.ds(OFF_BC, 16)], tmp_f)
    bc_vec = tmp_f[...]
    bstar = bc_vec[0].astype(jnp.int32)
    rem = bc_vec[1]

    def zero_fine(i, _):
      fcnt_v[pl.ds(i * 16, 16)] = zeros_f
      fsum_v[pl.ds(i * 16, 16)] = zeros_f
      return 0
    lax.fori_loop(0, (L * NB) // 16, zero_fine, 0)

    def scan_body(i, acc):
      dv = d_v[pl.ds(i * 16, 16)]
      u = plsc.bitcast(dv, jnp.int32)
      b = lax.shift_right_logical(u, COARSE_SHIFT)
      acc = acc + jnp.where(b > bstar, dv, jnp.float32(0))
      eq = b == bstar
      fb = jnp.bitwise_and(lax.shift_right_logical(u, FINE_SHIFT), NB - 1)
      idx = lane_base + fb
      plsc.addupdate_scatter(fcnt_v, [idx], ones_f, mask=eq)
      plsc.addupdate_scatter(fsum_v, [idx], dv, mask=eq)
      return acc
    acc = lax.fori_loop(0, PER_TILE // 16, scan_body, zeros_f)

    def merge_lanes_2(g, _):
      acc_c = zeros_f
      acc_s = zeros_f
      for l in range(L):
        acc_c = acc_c + fcnt_v[pl.ds(l * NB + g * 16, 16)]
        acc_s = acc_s + fsum_v[pl.ds(l * NB + g * 16, 16)]
      c1_v[pl.ds(g * 16, 16)] = acc_c
      cs_v[pl.ds(g * 16, 16)] = acc_s
      return 0
    lax.fori_loop(0, NB // 16, merge_lanes_2, 0)

    pltpu.sync_copy(c1_v, sh.at[pl.ds(OFF_CNT + s * NB, NB)])
    pltpu.sync_copy(cs_v, sh.at[pl.ds(OFF_SUM + s * NB, NB)])
    tmp_f[...] = acc
    pltpu.sync_copy(tmp_f, sh.at[pl.ds(OFF_ACC + s * L, L)])
    plsc.subcore_barrier()

    @pl.when(s == 0)
    def _():
      def zero_c1(i, _):
        c1_v[pl.ds(i * 16, 16)] = zeros_f
        cs_v[pl.ds(i * 16, 16)] = zeros_f
        return 0
      lax.fori_loop(0, NB // 16, zero_c1, 0)
      acc_all = zeros_f
      for t in range(NS):
        pltpu.sync_copy(sh.at[pl.ds(OFF_CNT + t * NB, NB)], row_c)
        pltpu.sync_copy(sh.at[pl.ds(OFF_SUM + t * NB, NB)], row_s)

        def add_row(g, _):
          c1_v[pl.ds(g * 16, 16)] = c1_v[pl.ds(g * 16, 16)] + row_c[pl.ds(g * 16, 16)]
          cs_v[pl.ds(g * 16, 16)] = cs_v[pl.ds(g * 16, 16)] + row_s[pl.ds(g * 16, 16)]
          return 0
        lax.fori_loop(0, NB // 16, add_row, 0)
        pltpu.sync_copy(sh.at[pl.ds(OFF_ACC + t * L, L)], tmp_f)
        acc_all = acc_all + tmp_f[...]
      s_above = jnp.sum(acc_all)

      fst, fcnt_ab, fsum_ab = _suffix_find(c1_v, cs_v, rem, lanes)
      leftover = rem - fcnt_ab
      vbits = (lax.shift_left(bstar, COARSE_SHIFT)
               | lax.shift_left(fst, FINE_SHIFT)
               | jnp.int32(1 << (FINE_SHIFT - 1)))
      vhat = jnp.max(plsc.bitcast(jnp.full((16,), vbits, jnp.int32),
                                  jnp.float32))
      s_slice = s_above + fsum_ab + leftover * vhat
      tmp_f[...] = jnp.full((16,), s_slice, jnp.float32)
      pltpu.sync_copy(tmp_f, out_hbm.at[slice_idx])
    plsc.subcore_barrier()
    return 0

  lax.fori_loop(0, SLICES_PER_CORE, per_slice, 0)


@jax.jit
def _run(x, y):
  mesh = plsc.VectorSubcoreMesh(core_axis_name="c", subcore_axis_name="s")
  fn = pl.kernel(
      _loss_program,
      out_type=jax.ShapeDtypeStruct((NSLICE, L), jnp.float32),
      mesh=mesh,
      compiler_params=pltpu.CompilerParams(needs_layout_passes=False),
      scratch_types=[
          pltpu.VMEM((PER_TILE,), jnp.float32),
          pltpu.VMEM((CHUNK,), jnp.float32),
          pltpu.VMEM((L * NB,), jnp.float32),
          pltpu.VMEM((L * NB,), jnp.float32),
          pltpu.VMEM((L * NB,), jnp.float32),
          pltpu.VMEM((NB,), jnp.float32),
          pltpu.VMEM((NB,), jnp.float32),
          pltpu.VMEM((NB,), jnp.float32),
          pltpu.VMEM((NB,), jnp.float32),
          pltpu.VMEM((16,), jnp.float32),
          pltpu.VMEM_SHARED((SH_WORDS,), jnp.float32),
      ],
  )
  return fn(x, y)


def kernel(input, label):
  x = input.reshape(-1)
  y = label.reshape(-1)
  out = _run(x, y)
  loss = jnp.sum(out[:, 0]) * jnp.float32(1.0 / (K * 4))
  return loss.reshape(1).astype(jnp.float32)


# speedup vs baseline: 21.7546x; 1.2267x over previous
import functools

import jax
import jax.numpy as jnp
from jax import lax
from jax.experimental import pallas as pl
from jax.experimental.pallas import tpu as pltpu
from jax.experimental.pallas import tpu_sc as plsc

NC = 2
NS = 16
L = 16
NSLICE = 8
NELEM = 1 << 20
K = 524288
PER_TILE = NELEM // NS
CHUNK = 8192
NCHUNK = PER_TILE // CHUNK
NB = 512
COARSE_SHIFT = 22
FINE_SHIFT = 13
SLICES_PER_CORE = NSLICE // NC
UNROLL = 8

OFF_CNT = 0
OFF_ACC = NS * NB
OFF_BC = NS * NB + NS * L
SH_WORDS = OFF_BC + 16


def _suffix_find(cnt_ref, sum_ref, thresh, lanes):
  def body(i, carry):
    found, bst, sac, sas, c_run, s_run = carry
    g = 31 - i
    c = cnt_ref[pl.ds(g * 16, 16)]
    s = sum_ref[pl.ds(g * 16, 16)]
    suffc = lax.rev(plsc.cumsum(lax.rev(c, (0,))), (0,))
    suffs = lax.rev(plsc.cumsum(lax.rev(s, (0,))), (0,))
    incl = c_run + suffc
    mask = incl >= thresh
    pc = jnp.max(plsc.all_reduce_population_count(mask))
    found_here = jnp.logical_and(pc > 0, jnp.logical_not(found))
    p0 = pc - 1
    onehot = lanes == p0
    cp = jnp.sum(jnp.where(onehot, c, jnp.float32(0)))
    sufcp = jnp.sum(jnp.where(onehot, suffc, jnp.float32(0)))
    sp = jnp.sum(jnp.where(onehot, s, jnp.float32(0)))
    sufsp = jnp.sum(jnp.where(onehot, suffs, jnp.float32(0)))
    bst = jnp.where(found_here, g * 16 + p0, bst)
    sac = jnp.where(found_here, c_run + sufcp - cp, sac)
    sas = jnp.where(found_here, s_run + sufsp - sp, sas)
    c_run = c_run + jnp.sum(c)
    s_run = s_run + jnp.sum(s)
    found = jnp.logical_or(found, pc > 0)
    return found, bst, sac, sas, c_run, s_run

  init = (jnp.bool_(False), jnp.int32(0), jnp.float32(0), jnp.float32(0),
          jnp.float32(0), jnp.float32(0))
  out = lax.fori_loop(0, 32, body, init)
  return out[1], out[2], out[3]


def _loss_program(x_hbm, y_hbm, out_hbm,
                  d_v, y_v, h1_v, fcnt_v,
                  c1_v, cs_v, row_c, tmp_f, sh,
                  sem_x0, sem_x1, sem_y0, sem_y1):
  c = lax.axis_index("c")
  s = lax.axis_index("s")
  lanes = jnp.arange(16, dtype=jnp.int32)
  lane_base = lanes * NB
  ones_f = jnp.ones((16,), jnp.float32)
  zeros_f = jnp.zeros((16,), jnp.float32)
  sems_x = (sem_x0, sem_x1)
  sems_y = (sem_y0, sem_y1)

  def per_slice(si, _):
    slice_idx = c * SLICES_PER_CORE + si
    base = slice_idx * NELEM + s * PER_TILE

    xcp = pltpu.async_copy(x_hbm.at[pl.ds(base, CHUNK)],
                           d_v.at[pl.ds(0, CHUNK)], sems_x[0])
    ycp = pltpu.async_copy(y_hbm.at[pl.ds(base, CHUNK)],
                           y_v.at[pl.ds(0, CHUNK)], sems_y[0])

    def zero_h1(i, _):
      for u in range(UNROLL):
        h1_v[pl.ds(i * (16 * UNROLL) + u * 16, 16)] = zeros_f
      return 0
    lax.fori_loop(0, (L * NB) // (16 * UNROLL), zero_h1, 0)

    for j in range(NCHUNK):
      par = j % 2
      npar = (j + 1) % 2
      if j + 1 < NCHUNK:
        xcp_n = pltpu.async_copy(
            x_hbm.at[pl.ds(base + (j + 1) * CHUNK, CHUNK)],
            d_v.at[pl.ds((j + 1) * CHUNK, CHUNK)], sems_x[npar])
        ycp_n = pltpu.async_copy(
            y_hbm.at[pl.ds(base + (j + 1) * CHUNK, CHUNK)],
            y_v.at[pl.ds(npar * CHUNK, CHUNK)], sems_y[npar])
      xcp.wait()
      ycp.wait()

      def vec_body(i, _):
        for u in range(UNROLL):
          off = j * CHUNK + i * (16 * UNROLL) + u * 16
          yoff = par * CHUNK + i * (16 * UNROLL) + u * 16
          xv = d_v[pl.ds(off, 16)]
          yv = y_v[pl.ds(yoff, 16)]
          dv = xv - yv
          dv = dv * dv
          d_v[pl.ds(off, 16)] = dv
          u32 = plsc.bitcast(dv, jnp.int32)
          b = lax.shift_right_logical(u32, COARSE_SHIFT)
          plsc.addupdate_scatter(h1_v, [lane_base + b], ones_f)
        return 0
      lax.fori_loop(0, CHUNK // (16 * UNROLL), vec_body, 0)
      if j + 1 < NCHUNK:
        xcp = xcp_n
        ycp = ycp_n

    def merge_lanes_1(g, _):
      acc = zeros_f
      for l in range(L):
        acc = acc + h1_v[pl.ds(l * NB + g * 16, 16)]
      c1_v[pl.ds(g * 16, 16)] = acc
      return 0
    lax.fori_loop(0, NB // 16, merge_lanes_1, 0)

    pltpu.sync_copy(c1_v, sh.at[pl.ds(OFF_CNT + s * NB, NB)])
    plsc.subcore_barrier()

    @pl.when(s == 0)
    def _():
      def zero_c1(i, _):
        c1_v[pl.ds(i * 16, 16)] = zeros_f
        cs_v[pl.ds(i * 16, 16)] = zeros_f
        return 0
      lax.fori_loop(0, NB // 16, zero_c1, 0)
      for t in range(NS):
        pltpu.sync_copy(sh.at[pl.ds(OFF_CNT + t * NB, NB)], row_c)

        def add_row(g, _):
          c1_v[pl.ds(g * 16, 16)] = c1_v[pl.ds(g * 16, 16)] + row_c[pl.ds(g * 16, 16)]
          return 0
        lax.fori_loop(0, NB // 16, add_row, 0)
      bst, sac, _ = _suffix_find(c1_v, cs_v, jnp.float32(K), lanes)
      rem = jnp.float32(K) - sac
      bc = (jnp.where(lanes == 0, bst.astype(jnp.float32), jnp.float32(0))
            + jnp.where(lanes == 1, rem, jnp.float32(0)))
      tmp_f[...] = bc
      pltpu.sync_copy(tmp_f, sh.at[pl.ds(OFF_BC, 16)])
    plsc.subcore_barrier()

    pltpu.sync_copy(sh.at[pl.ds(OFF_BC, 16)], tmp_f)
    bc_vec = tmp_f[...]
    bstar = bc_vec[0].astype(jnp.int32)
    rem = bc_vec[1]

    def zero_fine(i, _):
      for u in range(UNROLL):
        fcnt_v[pl.ds(i * (16 * UNROLL) + u * 16, 16)] = zeros_f
      return 0
    lax.fori_loop(0, (L * NB) // (16 * UNROLL), zero_fine, 0)

    def scan_body(i, acc):
      for u in range(UNROLL):
        off = i * (16 * UNROLL) + u * 16
        dv = d_v[pl.ds(off, 16)]
        u32 = plsc.bitcast(dv, jnp.int32)
        b = lax.shift_right_logical(u32, COARSE_SHIFT)
        acc = acc + jnp.where(b > bstar, dv, jnp.float32(0))
        eq = b == bstar
        fb = jnp.bitwise_and(lax.shift_right_logical(u32, FINE_SHIFT), NB - 1)
        plsc.addupdate_scatter(fcnt_v, [lane_base + fb], ones_f, mask=eq)
      return acc
    acc = lax.fori_loop(0, PER_TILE // (16 * UNROLL), scan_body, zeros_f)

    def merge_lanes_2(g, _):
      acc_c = zeros_f
      for l in range(L):
        acc_c = acc_c + fcnt_v[pl.ds(l * NB + g * 16, 16)]
      c1_v[pl.ds(g * 16, 16)] = acc_c
      return 0
    lax.fori_loop(0, NB // 16, merge_lanes_2, 0)

    pltpu.sync_copy(c1_v, sh.at[pl.ds(OFF_CNT + s * NB, NB)])
    tmp_f[...] = acc
    pltpu.sync_copy(tmp_f, sh.at[pl.ds(OFF_ACC + s * L, L)])
    plsc.subcore_barrier()

    @pl.when(s == 0)
    def _():
      def zero_c1(i, _):
        c1_v[pl.ds(i * 16, 16)] = zeros_f
        return 0
      lax.fori_loop(0, NB // 16, zero_c1, 0)
      acc_all = zeros_f
      for t in range(NS):
        pltpu.sync_copy(sh.at[pl.ds(OFF_CNT + t * NB, NB)], row_c)

        def add_row(g, _):
          c1_v[pl.ds(g * 16, 16)] = c1_v[pl.ds(g * 16, 16)] + row_c[pl.ds(g * 16, 16)]
          return 0
        lax.fori_loop(0, NB // 16, add_row, 0)
        pltpu.sync_copy(sh.at[pl.ds(OFF_ACC + t * L, L)], tmp_f)
        acc_all = acc_all + tmp_f[...]
      s_above = jnp.sum(acc_all)

      def synth(g, _):
        f = g * 16 + lanes
        vbits = (lax.shift_left(bstar, COARSE_SHIFT)
                 | lax.shift_left(f, FINE_SHIFT)
                 | jnp.int32(1 << (FINE_SHIFT - 1)))
        vals = plsc.bitcast(vbits, jnp.float32)
        cs_v[pl.ds(g * 16, 16)] = c1_v[pl.ds(g * 16, 16)] * vals
        return 0
      lax.fori_loop(0, NB // 16, synth, 0)

      fst, fcnt_ab, fsum_ab = _suffix_find(c1_v, cs_v, rem, lanes)
      leftover = rem - fcnt_ab
      vbits = (lax.shift_left(bstar, COARSE_SHIFT)
               | lax.shift_left(fst, FINE_SHIFT)
               | jnp.int32(1 << (FINE_SHIFT - 1)))
      vhat = jnp.max(plsc.bitcast(jnp.full((16,), vbits, jnp.int32),
                                  jnp.float32))
      s_slice = s_above + fsum_ab + leftover * vhat
      tmp_f[...] = jnp.full((16,), s_slice, jnp.float32)
      pltpu.sync_copy(tmp_f, out_hbm.at[slice_idx])
    plsc.subcore_barrier()
    return 0

  lax.fori_loop(0, SLICES_PER_CORE, per_slice, 0)


@jax.jit
def _run(x, y):
  mesh = plsc.VectorSubcoreMesh(core_axis_name="c", subcore_axis_name="s")
  fn = pl.kernel(
      _loss_program,
      out_type=jax.ShapeDtypeStruct((NSLICE, L), jnp.float32),
      mesh=mesh,
      compiler_params=pltpu.CompilerParams(needs_layout_passes=False),
      scratch_types=[
          pltpu.VMEM((PER_TILE,), jnp.float32),
          pltpu.VMEM((2 * CHUNK,), jnp.float32),
          pltpu.VMEM((L * NB,), jnp.float32),
          pltpu.VMEM((L * NB,), jnp.float32),
          pltpu.VMEM((NB,), jnp.float32),
          pltpu.VMEM((NB,), jnp.float32),
          pltpu.VMEM((NB,), jnp.float32),
          pltpu.VMEM((16,), jnp.float32),
          pltpu.VMEM_SHARED((SH_WORDS,), jnp.float32),
          pltpu.SemaphoreType.DMA,
          pltpu.SemaphoreType.DMA,
          pltpu.SemaphoreType.DMA,
          pltpu.SemaphoreType.DMA,
      ],
  )
  return fn(x, y)


def kernel(input, label):
  x = input.reshape(-1)
  y = label.reshape(-1)
  out = _run(x, y)
  loss = jnp.sum(out[:, 0]) * jnp.float32(1.0 / (K * 4))
  return loss.reshape(1).astype(jnp.float32)


# speedup vs baseline: 22.9344x; 1.0542x over previous
import functools

import jax
import jax.numpy as jnp
from jax import lax
from jax.experimental import pallas as pl
from jax.experimental.pallas import tpu as pltpu
from jax.experimental.pallas import tpu_sc as plsc

NC = 2
NS = 16
L = 16
NSLICE = 8
NELEM = 1 << 20
K = 524288
PER_TILE = NELEM // NS
CHUNK = 8192
NCHUNK = PER_TILE // CHUNK
NB1 = 256
NB2 = 512
SUB1 = 4
SUB2 = 2
COARSE_SHIFT = 23
FINE_SHIFT = 14
SLICES_PER_CORE = NSLICE // NC
UNROLL = 8
H1_WORDS = SUB1 * L * NB1
F2_WORDS = SUB2 * L * NB2

OFF_C1 = 0
OFF_C2 = NS * NB1
OFF_ACC = OFF_C2 + NS * NB2
OFF_BC = OFF_ACC + NS * L
SH_WORDS = OFF_BC + 16


def _suffix_find(cnt_ref, sum_ref, thresh, lanes, ngroups):
  def body(i, carry):
    found, bst, sac, sas, c_run, s_run = carry
    g = ngroups - 1 - i
    c = cnt_ref[pl.ds(g * 16, 16)]
    s = sum_ref[pl.ds(g * 16, 16)]
    suffc = lax.rev(plsc.cumsum(lax.rev(c, (0,))), (0,))
    suffs = lax.rev(plsc.cumsum(lax.rev(s, (0,))), (0,))
    incl = c_run + suffc
    mask = incl >= thresh
    pc = jnp.max(plsc.all_reduce_population_count(mask))
    found_here = jnp.logical_and(pc > 0, jnp.logical_not(found))
    p0 = pc - 1
    onehot = lanes == p0
    cp = jnp.sum(jnp.where(onehot, c, jnp.float32(0)))
    sufcp = jnp.sum(jnp.where(onehot, suffc, jnp.float32(0)))
    sp = jnp.sum(jnp.where(onehot, s, jnp.float32(0)))
    sufsp = jnp.sum(jnp.where(onehot, suffs, jnp.float32(0)))
    bst = jnp.where(found_here, g * 16 + p0, bst)
    sac = jnp.where(found_here, c_run + sufcp - cp, sac)
    sas = jnp.where(found_here, s_run + sufsp - sp, sas)
    c_run = c_run + jnp.sum(c)
    s_run = s_run + jnp.sum(s)
    found = jnp.logical_or(found, pc > 0)
    return found, bst, sac, sas, c_run, s_run

  init = (jnp.bool_(False), jnp.int32(0), jnp.float32(0), jnp.float32(0),
          jnp.float32(0), jnp.float32(0))
  out = lax.fori_loop(0, ngroups, body, init)
  return out[1], out[2], out[3]


def _loss_program(x_hbm, y_hbm, out_hbm,
                  d_v, y_v, h1_v, fcnt_v,
                  c1_v, cs_v, tmp_f, sh,
                  sem_x0, sem_x1, sem_y0, sem_y1):
  c = lax.axis_index("c")
  s = lax.axis_index("s")
  lanes = jnp.arange(16, dtype=jnp.int32)
  base1 = [lanes * NB1 + r * (L * NB1) for r in range(SUB1)]
  base2 = [lanes * NB2 + r * (L * NB2) for r in range(SUB2)]
  ones_f = jnp.ones((16,), jnp.float32)
  zeros_f = jnp.zeros((16,), jnp.float32)
  sems_x = (sem_x0, sem_x1)
  sems_y = (sem_y0, sem_y1)

  def per_slice(si, _):
    slice_idx = c * SLICES_PER_CORE + si
    base = slice_idx * NELEM + s * PER_TILE

    xcp = pltpu.async_copy(x_hbm.at[pl.ds(base, CHUNK)],
                           d_v.at[pl.ds(0, CHUNK)], sems_x[0])
    ycp = pltpu.async_copy(y_hbm.at[pl.ds(base, CHUNK)],
                           y_v.at[pl.ds(0, CHUNK)], sems_y[0])

    def zero_h1(i, _):
      for u in range(UNROLL):
        h1_v[pl.ds(i * (16 * UNROLL) + u * 16, 16)] = zeros_f
      return 0
    lax.fori_loop(0, H1_WORDS // (16 * UNROLL), zero_h1, 0)

    for j in range(NCHUNK):
      par = j % 2
      npar = (j + 1) % 2
      if j + 1 < NCHUNK:
        xcp_n = pltpu.async_copy(
            x_hbm.at[pl.ds(base + (j + 1) * CHUNK, CHUNK)],
            d_v.at[pl.ds((j + 1) * CHUNK, CHUNK)], sems_x[npar])
        ycp_n = pltpu.async_copy(
            y_hbm.at[pl.ds(base + (j + 1) * CHUNK, CHUNK)],
            y_v.at[pl.ds(npar * CHUNK, CHUNK)], sems_y[npar])
      xcp.wait()
      ycp.wait()

      def vec_body(i, _):
        for u in range(UNROLL):
          off = j * CHUNK + i * (16 * UNROLL) + u * 16
          yoff = par * CHUNK + i * (16 * UNROLL) + u * 16
          xv = d_v[pl.ds(off, 16)]
          yv = y_v[pl.ds(yoff, 16)]
          dv = xv - yv
          dv = dv * dv
          d_v[pl.ds(off, 16)] = dv
          u32 = plsc.bitcast(dv, jnp.int32)
          b = lax.shift_right_logical(u32, COARSE_SHIFT)
          plsc.addupdate_scatter(h1_v, [base1[u % SUB1] + b], ones_f)
        return 0
      lax.fori_loop(0, CHUNK // (16 * UNROLL), vec_body, 0)
      if j + 1 < NCHUNK:
        xcp = xcp_n
        ycp = ycp_n

    def merge_lanes_1(g, _):
      acc = zeros_f
      for r in range(SUB1):
        for l in range(L):
          acc = acc + h1_v[pl.ds(r * (L * NB1) + l * NB1 + g * 16, 16)]
      c1_v[pl.ds(g * 16, 16)] = acc
      return 0
    lax.fori_loop(0, NB1 // 16, merge_lanes_1, 0)

    pltpu.sync_copy(c1_v.at[pl.ds(0, NB1)], sh.at[pl.ds(OFF_C1 + s * NB1, NB1)])
    plsc.subcore_barrier()

    @pl.when(s == 0)
    def _():
      pltpu.sync_copy(sh.at[pl.ds(OFF_C1, NS * NB1)],
                      fcnt_v.at[pl.ds(0, NS * NB1)])

      def zero_cs(i, _):
        cs_v[pl.ds(i * 16, 16)] = zeros_f
        return 0
      lax.fori_loop(0, NB1 // 16, zero_cs, 0)

      def merge_rows_1(g, _):
        acc = zeros_f
        for t in range(NS):
          acc = acc + fcnt_v[pl.ds(t * NB1 + g * 16, 16)]
        c1_v[pl.ds(g * 16, 16)] = acc
        return 0
      lax.fori_loop(0, NB1 // 16, merge_rows_1, 0)

      bst, sac, _ = _suffix_find(c1_v, cs_v, jnp.float32(K), lanes, NB1 // 16)
      rem = jnp.float32(K) - sac
      bc = (jnp.where(lanes == 0, bst.astype(jnp.float32), jnp.float32(0))
            + jnp.where(lanes == 1, rem, jnp.float32(0)))
      tmp_f[...] = bc
      pltpu.sync_copy(tmp_f, sh.at[pl.ds(OFF_BC, 16)])
    plsc.subcore_barrier()

    pltpu.sync_copy(sh.at[pl.ds(OFF_BC, 16)], tmp_f)
    bc_vec = tmp_f[...]
    bstar = bc_vec[0].astype(jnp.int32)
    rem = bc_vec[1]

    def zero_fine(i, _):
      for u in range(UNROLL):
        fcnt_v[pl.ds(i * (16 * UNROLL) + u * 16, 16)] = zeros_f
      return 0
    lax.fori_loop(0, F2_WORDS // (16 * UNROLL), zero_fine, 0)

    def scan_body(i, carry):
      acc0, acc1 = carry
      for u in range(UNROLL):
        off = i * (16 * UNROLL) + u * 16
        dv = d_v[pl.ds(off, 16)]
        u32 = plsc.bitcast(dv, jnp.int32)
        b = lax.shift_right_logical(u32, COARSE_SHIFT)
        contrib = jnp.where(b > bstar, dv, jnp.float32(0))
        if u % 2 == 0:
          acc0 = acc0 + contrib
        else:
          acc1 = acc1 + contrib
        eq = b == bstar
        fb = jnp.bitwise_and(lax.shift_right_logical(u32, FINE_SHIFT), NB2 - 1)
        plsc.addupdate_scatter(fcnt_v, [base2[u % SUB2] + fb], ones_f, mask=eq)
      return acc0, acc1
    acc0, acc1 = lax.fori_loop(0, PER_TILE // (16 * UNROLL), scan_body,
                               (zeros_f, zeros_f))
    acc = acc0 + acc1

    def merge_lanes_2(g, _):
      acc_c = zeros_f
      for r in range(SUB2):
        for l in range(L):
          acc_c = acc_c + fcnt_v[pl.ds(r * (L * NB2) + l * NB2 + g * 16, 16)]
      c1_v[pl.ds(g * 16, 16)] = acc_c
      return 0
    lax.fori_loop(0, NB2 // 16, merge_lanes_2, 0)

    pltpu.sync_copy(c1_v, sh.at[pl.ds(OFF_C2 + s * NB2, NB2)])
    tmp_f[...] = acc
    pltpu.sync_copy(tmp_f, sh.at[pl.ds(OFF_ACC + s * L, L)])
    plsc.subcore_barrier()

    @pl.when(s == 0)
    def _():
      pltpu.sync_copy(sh.at[pl.ds(OFF_C2, NS * NB2)],
                      h1_v.at[pl.ds(0, NS * NB2)])
      pltpu.sync_copy(sh.at[pl.ds(OFF_ACC, NS * L)],
                      h1_v.at[pl.ds(NS * NB2, NS * L)])

      def merge_rows_2(g, _):
        acc_c = zeros_f
        for t in range(NS):
          acc_c = acc_c + h1_v[pl.ds(t * NB2 + g * 16, 16)]
        c1_v[pl.ds(g * 16, 16)] = acc_c
        return 0
      lax.fori_loop(0, NB2 // 16, merge_rows_2, 0)

      acc_all = zeros_f
      for t in range(NS):
        acc_all = acc_all + h1_v[pl.ds(NS * NB2 + t * L, 16)]
      s_above = jnp.sum(acc_all)

      def synth(g, _):
        f = g * 16 + lanes
        vbits = (lax.shift_left(bstar, COARSE_SHIFT)
                 | lax.shift_left(f, FINE_SHIFT)
                 | jnp.int32(1 << (FINE_SHIFT - 1)))
        vals = plsc.bitcast(vbits, jnp.float32)
        cs_v[pl.ds(g * 16, 16)] = c1_v[pl.ds(g * 16, 16)] * vals
        return 0
      lax.fori_loop(0, NB2 // 16, synth, 0)

      fst, fcnt_ab, fsum_ab = _suffix_find(c1_v, cs_v, rem, lanes, NB2 // 16)
      leftover = rem - fcnt_ab
      vbits = (lax.shift_left(bstar, COARSE_SHIFT)
               | lax.shift_left(fst, FINE_SHIFT)
               | jnp.int32(1 << (FINE_SHIFT - 1)))
      vhat = jnp.max(plsc.bitcast(jnp.full((16,), vbits, jnp.int32),
                                  jnp.float32))
      s_slice = s_above + fsum_ab + leftover * vhat
      tmp_f[...] = jnp.full((16,), s_slice, jnp.float32)
      pltpu.sync_copy(tmp_f, out_hbm.at[slice_idx])
    return 0

  lax.fori_loop(0, SLICES_PER_CORE, per_slice, 0)


@jax.jit
def _run(x, y):
  mesh = plsc.VectorSubcoreMesh(core_axis_name="c", subcore_axis_name="s")
  fn = pl.kernel(
      _loss_program,
      out_type=jax.ShapeDtypeStruct((NSLICE, L), jnp.float32),
      mesh=mesh,
      compiler_params=pltpu.CompilerParams(needs_layout_passes=False),
      scratch_types=[
          pltpu.VMEM((PER_TILE,), jnp.float32),
          pltpu.VMEM((2 * CHUNK,), jnp.float32),
          pltpu.VMEM((H1_WORDS,), jnp.float32),
          pltpu.VMEM((F2_WORDS,), jnp.float32),
          pltpu.VMEM((NB2,), jnp.float32),
          pltpu.VMEM((NB2,), jnp.float32),
          pltpu.VMEM((16,), jnp.float32),
          pltpu.VMEM_SHARED((SH_WORDS,), jnp.float32),
          pltpu.SemaphoreType.DMA,
          pltpu.SemaphoreType.DMA,
          pltpu.SemaphoreType.DMA,
          pltpu.SemaphoreType.DMA,
      ],
  )
  return fn(x, y)


def kernel(input, label):
  x = input.reshape(-1)
  y = label.reshape(-1)
  out = _run(x, y)
  loss = jnp.sum(out[:, 0]) * jnp.float32(1.0 / (K * 4))
  return loss.reshape(1).astype(jnp.float32)
